# Initial kernel scaffold; baseline (speedup 1.0000x reference)
import jax
import jax.numpy as jnp
from jax.experimental import pallas as pl


def kernel(input):
    raise NotImplementedError("write your pallas kernel here")


# speedup vs baseline: 1.2949x; 1.2949x over previous
import functools

import jax
import jax.numpy as jnp
from jax import lax
from jax.experimental import pallas as pl
from jax.experimental.pallas import tpu as pltpu
from jax.experimental.pallas import tpu_sc as plsc

I32 = jnp.int32
F32 = jnp.float32

NT = 16
N = 200000
NPAD = 204800
CHUNK = 12800
W = 1280
WEL = W * 4
NWIN = CHUNK // W
WGRP = W // 16
WPAIR = W // 32

NBINS = 50000
JUNKBIN = NBINS
NBINSP = 51200
SLICE = NBINSP // NT
SLICEV = SLICE // 16

MAXV = 30000
MAXP = 35
VROWS = MAXV * MAXP
JROW = VROWS
VROWSP = 1050112
VELS = VROWSP * 4
ZEL = VELS // NT
NZW = ZEL // WEL
ZTAIL = ZEL - NZW * WEL
CNTP = 30208
CORP = 30208


def _build():
    mesh = plsc.VectorSubcoreMesh(
        core_axis_name="c", subcore_axis_name="s", num_cores=1
    )

    @functools.partial(
        pl.kernel,
        mesh=mesh,
        compiler_params=pltpu.CompilerParams(needs_layout_passes=False),
        out_type=[
            jax.ShapeDtypeStruct((VELS,), F32),
            jax.ShapeDtypeStruct((CORP * 3,), I32),
            jax.ShapeDtypeStruct((CNTP,), I32),
        ],
        scratch_types=[
            pltpu.VMEM((WEL,), F32),
            pltpu.VMEM((WEL,), F32),
            pltpu.VMEM((CHUNK,), I32),
            pltpu.VMEM((CHUNK,), I32),
            pltpu.VMEM((NBINSP,), I32),
            pltpu.VMEM((SLICE,), I32),
            pltpu.VMEM((CHUNK + 16,), I32),
            pltpu.VMEM((W,), I32),
            pltpu.VMEM((WEL,), I32),
            pltpu.VMEM((32,), I32),
            pltpu.VMEM((SLICE,), I32),
            pltpu.VMEM((SLICE,), I32),
            pltpu.VMEM((SLICE,), I32),
            pltpu.VMEM((128,), I32),
            pltpu.VMEM((128,), I32),
            pltpu.VMEM((16,), I32),
            pltpu.VMEM((16, 16), I32),
            pltpu.HBM((NT, NBINSP), I32),
            pltpu.VMEM_SHARED((NBINSP,), I32),
            pltpu.VMEM_SHARED((NT, 16), I32),
            pltpu.VMEM_SHARED((NT, 16), I32),
            pltpu.SemaphoreType.DMA,
            pltpu.SemaphoreType.DMA,
            pltpu.SemaphoreType.DMA,
            pltpu.SemaphoreType.DMA,
        ],
    )
    def vox_kernel(pts, voxf, corf, cntf,
                   wb0, wb1, flatb, rankb, histb, totb, binsb, voxb, dstE,
                   dtmp, ssl, idxb, valb, ib2, vb2, nfb, rdb,
                   H, slotT, NFS, NFU, semZ, semIn, semOut, semG):
        t = lax.axis_index("s")
        iota = lax.iota(I32, 16)
        zero16 = jnp.zeros((16,), I32)
        zf32 = jnp.zeros((16,), F32)
        mv16 = jnp.full((16,), MAXV, I32)
        one16 = jnp.full((16,), 1, I32)
        pbase = t * CHUNK

        def zero_wb0(i, c):
            wb0[pl.ds(i * 16, 16)] = zf32
            return c
        lax.fori_loop(0, WEL // 16, zero_wb0, 0)

        zel0 = t * ZEL

        def fire_z(i, c):
            for b in range(4):
                pltpu.async_copy(
                    wb0, voxf.at[pl.ds(zel0 + (i * 4 + b) * WEL, WEL)], semZ)
            for b in range(4):
                pltpu.make_async_copy(
                    wb0, voxf.at[pl.ds(0, WEL)], semZ).wait()
            return c
        lax.fori_loop(0, NZW // 4, fire_z, 0)
        for i in range(NZW - (NZW // 4) * 4):
            pltpu.async_copy(
                wb0,
                voxf.at[pl.ds(zel0 + ((NZW // 4) * 4 + i) * WEL, WEL)],
                semZ)
        if ZTAIL:
            pltpu.async_copy(
                wb0.at[pl.ds(0, ZTAIL)],
                voxf.at[pl.ds(zel0 + NZW * WEL, ZTAIL)],
                semZ,
            )

        def zero_binsb(i, c):
            binsb[pl.ds(i * 16, 16)] = zero16
            return c
        lax.fori_loop(0, (CNTP // NT) // 16, zero_binsb, 0)
        pltpu.async_copy(
            binsb.at[pl.ds(0, CNTP // NT)],
            cntf.at[pl.ds(t * (CNTP // NT), CNTP // NT)],
            semZ,
        )

        def fill_ssl(i, c):
            ssl[pl.ds(i * 16, 16)] = mv16
            return c
        lax.fori_loop(0, SLICE // 16, fill_ssl, 0)
        pltpu.async_copy(ssl, slotT.at[pl.ds(t * SLICE, SLICE)], semZ)

        def zero_hist(i, c):
            histb[pl.ds(i * 16, 16)] = zero16
            return c
        lax.fori_loop(0, NBINSP // 16, zero_hist, 0)

        pltpu.async_copy(pts.at[pl.ds(pbase * 4, WEL)], wb1, semIn)

        for _ in range(NZW - (NZW // 4) * 4):
            pltpu.make_async_copy(
                wb0, voxf.at[pl.ds(0, WEL)], semZ).wait()
        if ZTAIL:
            pltpu.make_async_copy(
                wb0.at[pl.ds(0, ZTAIL)],
                voxf.at[pl.ds(0, ZTAIL)], semZ).wait()
        pltpu.make_async_copy(
            binsb.at[pl.ds(0, CNTP // NT)],
            cntf.at[pl.ds(0, CNTP // NT)], semZ).wait()
        pltpu.make_async_copy(
            ssl, slotT.at[pl.ds(0, SLICE)], semZ).wait()

        vs_xy = jnp.float32(0.02)
        vs_z = jnp.float32(0.05)
        g1000 = jnp.int32(1000)

        for w in range(NWIN):
            buf = wb1 if w % 2 == 0 else wb0
            nbuf = wb0 if w % 2 == 0 else wb1
            pltpu.make_async_copy(
                pts.at[pl.ds(0, WEL)], buf, semIn).wait()
            if w + 1 < NWIN:
                pltpu.async_copy(
                    pts.at[pl.ds((pbase + (w + 1) * W) * 4, WEL)], nbuf,
                    semIn)

            def p1_body(g, c, buf=buf, w=w):
                el = (g * 16 + iota) * 4
                px = plsc.load_gather(buf.at[:], [el])
                py = plsc.load_gather(buf.at[:], [el + 1])
                pz = plsc.load_gather(buf.at[:], [el + 2])
                ix = (px / vs_xy).astype(I32)
                iy = (py / vs_xy).astype(I32)
                iz = (pz / vs_z).astype(I32)
                valid = (
                    (px >= 0.0) & (py >= 0.0) & (pz >= 0.0)
                    & (ix < 50) & (iy < 50) & (iz < 20)
                )
                fl = jnp.where(valid, ix * g1000 + iy * 20 + iz,
                               jnp.int32(JUNKBIN))
                flatb[pl.ds(w * W + g * 16, 16)] = fl
                d, lastm = plsc.scan_count(fl)
                h = plsc.load_gather(histb.at[:], [fl])
                rankb[pl.ds(w * W + g * 16, 16)] = h + d - 1
                plsc.store_scatter(histb.at[:], [fl], h + d, mask=lastm)
                return c
            lax.fori_loop(0, WGRP, p1_body, 0)

        pltpu.sync_copy(histb, H.at[t])
        plsc.subcore_barrier()

        sbase = t * SLICE
        for ub in range(2):
            for u in range(ub * 8, ub * 8 + 8):
                pltpu.async_copy(
                    H.at[u, pl.ds(sbase, SLICE)],
                    histb.at[pl.ds(u * SLICE, SLICE)], semG)
            for u in range(ub * 8, ub * 8 + 8):
                pltpu.make_async_copy(
                    H.at[u, pl.ds(sbase, SLICE)],
                    histb.at[pl.ds(u * SLICE, SLICE)], semG).wait()

        def p3_body(j, nun):
            run = zero16
            for u in range(NT):
                x = histb[pl.ds(u * SLICE + j * 16, 16)]
                histb[pl.ds(u * SLICE + j * 16, 16)] = run
                run = run + x
            totb[pl.ds(j * 16, 16)] = run
            bing = sbase + j * 16 + iota
            m = (run == 0) & (bing < NBINS)
            return nun + jnp.sum(m.astype(I32))
        nunocc = lax.fori_loop(0, SLICEV, p3_body, jnp.int32(0))

        for ub in range(2):
            for u in range(ub * 8, ub * 8 + 8):
                pltpu.async_copy(
                    histb.at[pl.ds(u * SLICE, SLICE)],
                    H.at[u, pl.ds(sbase, SLICE)], semG)
            for u in range(ub * 8, ub * 8 + 8):
                pltpu.make_async_copy(
                    histb.at[pl.ds(u * SLICE, SLICE)],
                    H.at[u, pl.ds(sbase, SLICE)], semG).wait()

        nfb[...] = one16 * nunocc
        pltpu.sync_copy(nfb, NFU.at[t])
        plsc.subcore_barrier()

        pltpu.sync_copy(H.at[t], histb)

        def p4_body(g, foff):
            fl = flatb[pl.ds(g * 16, 16)]
            lr = rankb[pl.ds(g * 16, 16)]
            b = plsc.load_gather(histb.at[:], [fl])
            r = lr + b
            rankb[pl.ds(g * 16, 16)] = r
            isf = (r == 0) & (fl < NBINS)
            plsc.store_compressed(binsb.at[pl.ds(foff, 16)], fl, mask=isf)
            return foff + jnp.sum(isf.astype(I32))
        nfirst = lax.fori_loop(0, CHUNK // 16, p4_body, jnp.int32(0))

        nfb[...] = one16 * nfirst
        pltpu.sync_copy(nfb, NFS.at[t])
        plsc.subcore_barrier()

        pltpu.sync_copy(NFS, rdb)
        nfv = plsc.load_gather(rdb.at[:, :], [iota, zero16])
        firstbase = jnp.sum(jnp.where(iota < t, nfv, 0))
        n_occ = jnp.sum(nfv)
        pltpu.sync_copy(NFU, rdb)
        nuv = plsc.load_gather(rdb.at[:, :], [iota, zero16])
        unoccbase = jnp.sum(jnp.where(iota < t, nuv, 0))

        nsub = (nfirst + 127) // 128

        def p5_body(s, c):
            for k in range(8):
                lane = s * 128 + k * 16 + iota
                bv = binsb[pl.ds(s * 128 + k * 16, 16)]
                m = lane < nfirst
                padbin = jnp.int32(NBINS) + (lane % 1200)
                idxv = jnp.where(m, bv, padbin)
                slotv = jnp.minimum(firstbase + lane, MAXV)
                vv = jnp.where(m, slotv, jnp.int32(MAXV))
                ib2[pl.ds(k * 16, 16)] = idxv
                vb2[pl.ds(k * 16, 16)] = vv
            pltpu.sync_copy(vb2, slotT.at[ib2])
            return c
        lax.fori_loop(0, nsub, p5_body, 0)
        plsc.subcore_barrier()

        pltpu.sync_copy(slotT.at[pl.ds(sbase, SLICE)], ssl)

        def p6a(j, c):
            sv = ssl[pl.ds(j * 16, 16)]
            tv = totb[pl.ds(j * 16, 16)]
            off = j * 16
            occ = sv < MAXV
            crow = jnp.where(occ, sv,
                             jnp.int32(MAXV) + ((off + iota) % 208))
            idxb[pl.ds(off, 16)] = crow
            valb[pl.ds(off, 16)] = jnp.where(occ, jnp.minimum(tv, MAXP), 0)
            return c
        lax.fori_loop(0, SLICEV, p6a, 0)
        pltpu.sync_copy(valb, cntf.at[idxb])

        for comp in range(3):
            def p6b(j, c, comp=comp):
                sv = ssl[pl.ds(j * 16, 16)]
                off = j * 16
                bing = sbase + off + iota
                occ = sv < MAXV
                crow = jnp.where(occ, sv,
                                 jnp.int32(MAXV) + ((off + iota) % 208))
                idxb[pl.ds(off, 16)] = crow * 3 + comp
                if comp == 0:
                    cv = bing % 20
                elif comp == 1:
                    cv = (bing // 20) % 50
                else:
                    cv = bing // g1000
                valb[pl.ds(off, 16)] = cv
                return c
            lax.fori_loop(0, SLICEV, p6b, 0)
            pltpu.sync_copy(valb, corf.at[idxb])

        @pl.when(n_occ < MAXV)
        def _():
            fb = n_occ + unoccbase

            def p7a(j, uoff):
                tv = totb[pl.ds(j * 16, 16)]
                bing = sbase + j * 16 + iota
                m = (tv == 0) & (bing < NBINS)
                plsc.store_compressed(
                    binsb.at[pl.ds(uoff, 16)], bing, mask=m)
                return uoff + jnp.sum(m.astype(I32))
            uoff = lax.fori_loop(0, SLICEV, p7a, jnp.int32(0))

            for comp in range(3):
                def p7b(j, c, comp=comp):
                    lane = j * 16 + iota
                    bv = binsb[pl.ds(j * 16, 16)]
                    slotv = fb + lane
                    ok = (lane < uoff) & (slotv < MAXV)
                    crow = jnp.where(ok, slotv,
                                     jnp.int32(MAXV) + (lane % 208))
                    idxb[pl.ds(j * 16, 16)] = crow * 3 + comp
                    if comp == 0:
                        cv = bv % 20
                    elif comp == 1:
                        cv = (bv // 20) % 50
                    else:
                        cv = bv // g1000
                    valb[pl.ds(j * 16, 16)] = cv
                    return c
                lax.fori_loop(0, SLICEV, p7b, 0)
                pltpu.sync_copy(valb, corf.at[idxb])

        pltpu.async_copy(pts.at[pl.ds(pbase * 4, WEL)], wb1, semIn)
        j4 = iota // 4
        c4 = iota % 4
        for w in range(NWIN):
            buf = wb1 if w % 2 == 0 else wb0
            nbuf = wb0 if w % 2 == 0 else wb1
            pltpu.make_async_copy(
                pts.at[pl.ds(0, WEL)], buf, semIn).wait()
            if w + 1 < NWIN:
                pltpu.async_copy(
                    pts.at[pl.ds((pbase + (w + 1) * W) * 4, WEL)], nbuf,
                    semIn)
            pltpu.sync_copy(
                slotT.at[flatb.at[pl.ds(w * W, W)]], voxb)

            def p8_body(gg, c, w=w):
                for h in range(2):
                    g = gg * 2 + h
                    v = voxb[pl.ds(g * 16, 16)]
                    r = rankb[pl.ds(w * W + g * 16, 16)]
                    keep = (v < MAXV) & (r < MAXP)
                    d = jnp.where(keep, v * MAXP + r,
                                  jnp.int32(JROW) + ((g * 16 + iota) % 112))
                    dtmp[pl.ds(h * 16, 16)] = d
                for k in range(8):
                    dsel = plsc.load_gather(dtmp.at[:], [k * 4 + j4])
                    idxv = dsel * 4 + c4
                    dstE[pl.ds(gg * 128 + k * 16, 16)] = idxv
                return c
            lax.fori_loop(0, WPAIR, p8_body, 0)

            pltpu.async_copy(buf, voxf.at[dstE], semOut)
            pltpu.make_async_copy(buf, voxf.at[dstE], semOut).wait()

    return vox_kernel


def kernel(input):
    pts_pad = jnp.concatenate(
        [input.reshape(-1), jnp.full(((NPAD - N) * 4,), -1.0, input.dtype)]
    )
    voxf, corf, cntf = _build()(pts_pad)
    voxels = voxf[: VROWS * 4].reshape(MAXV, MAXP, 4)
    coors = corf[: MAXV * 3].reshape(MAXV, 3)
    return voxels, coors, cntf[:MAXV]
